# baseline (speedup 1.0000x reference)
import functools

import jax
import jax.numpy as jnp
from jax import lax
from jax.experimental import pallas as pl
from jax.experimental.pallas import tpu as pltpu
from jax.experimental.pallas import tpu_sc as plsc

_VOCAB = 100000
_D = 768
_N_TOK = 10
_B = 4
_S = 2048

_NC = 2
_NS = 16
_NW = _NC * _NS
_L = 16

_ROWS = _B * _S
_RPW = _ROWS // _NW
_NBURST = _RPW // _L
_NSLOT = 8
_WPB = _S // _RPW

_mesh = plsc.VectorSubcoreMesh(core_axis_name="c", subcore_axis_name="s")


@functools.partial(
    pl.kernel,
    mesh=_mesh,
    out_type=jax.ShapeDtypeStruct((_ROWS, _D), jnp.float32),
    scratch_types=[
        pltpu.VMEM((_B, _S), jnp.int32),
        pltpu.VMEM((_NSLOT, _L, _D), jnp.float32),
        pltpu.VMEM((_L, _D), jnp.float32),
        pltpu.SemaphoreType.DMA,
        pltpu.SemaphoreType.DMA,
        pltpu.SemaphoreType.DMA,
    ],
)
def _soft_embed(tokens_hbm, wte_hbm, learned_hbm, out_hbm,
                tok_v, rows_v, learned_v, gsem, osem, lsem):
    wid = lax.axis_index("s") * _NC + lax.axis_index("c")
    base = wid * _RPW
    b = wid // _WPB
    s0 = (wid % _WPB) * _RPW
    batch_start = base % _S == 0
    liota = jnp.minimum(lax.iota(jnp.int32, _L), _N_TOK - 1)

    pltpu.sync_copy(tokens_hbm, tok_v)

    @pl.when(batch_start)
    def _():
        pltpu.async_copy(learned_hbm.at[liota], learned_v, lsem).wait()

    gds = [None] * _NSLOT
    wds = [None] * _NSLOT

    def fire(i):
        slot = i % _NSLOT
        if wds[slot] is not None:
            wds[slot].wait()
            wds[slot] = None
        vidx = tok_v[b, pl.ds(s0 + i * _L, _L)]
        gds[slot] = pltpu.async_copy(wte_hbm.at[vidx], rows_v.at[slot], gsem)

    def drain(i):
        slot = i % _NSLOT
        gds[slot].wait()
        wds[slot] = pltpu.async_copy(
            rows_v.at[slot], out_hbm.at[pl.ds(base + i * _L, _L)], osem)

    for i in range(_NBURST):
        fire(i)
        j = i - (_NSLOT - 1)
        if j >= 0:
            drain(j)
    for j in range(_NBURST - (_NSLOT - 1), _NBURST):
        drain(j)
    for w in wds:
        if w is not None:
            w.wait()

    @pl.when(batch_start)
    def _():
        svidx = liota + b * _S
        pltpu.async_copy(learned_v, out_hbm.at[svidx], lsem).wait()


def kernel(tokens, wte, learned_embedding):
    out = _soft_embed(tokens, wte, learned_embedding)
    return out.reshape(_B, _S, _D)


# speedup vs baseline: 1.0329x; 1.0329x over previous
import functools

import jax
import jax.numpy as jnp
from jax import lax
from jax.experimental import pallas as pl
from jax.experimental.pallas import tpu as pltpu
from jax.experimental.pallas import tpu_sc as plsc

_VOCAB = 100000
_D = 768
_N_TOK = 10
_B = 4
_S = 2048

_NC = 2
_NS = 16
_NW = _NC * _NS
_L = 16

_ROWS = _B * _S
_RPW = _ROWS // _NW
_NBURST = _RPW // _L
_NSLOT = 8
_WPB = _S // _RPW

_mesh = plsc.VectorSubcoreMesh(core_axis_name="c", subcore_axis_name="s")


@functools.partial(
    pl.kernel,
    mesh=_mesh,
    out_type=jax.ShapeDtypeStruct((_ROWS, _D), jnp.float32),
    scratch_types=[
        pltpu.VMEM((_B, _S), jnp.int32),
        pltpu.VMEM((_NSLOT, _L, _D), jnp.float32),
        pltpu.VMEM((_L, _D), jnp.float32),
        pltpu.SemaphoreType.DMA,
        pltpu.SemaphoreType.DMA,
        pltpu.SemaphoreType.DMA,
    ],
)
def _soft_embed(tokens_hbm, wte_hbm, learned_hbm, out_hbm,
                tok_v, rows_v, learned_v, gsem, osem, lsem):
    wid = lax.axis_index("s") * _NC + lax.axis_index("c")
    base = wid * _RPW
    b = wid // _WPB
    s0 = (wid % _WPB) * _RPW
    batch_start = base % _S == 0
    liota = jnp.minimum(lax.iota(jnp.int32, _L), _N_TOK - 1)

    pltpu.sync_copy(tokens_hbm, tok_v)

    ldesc = []

    @pl.when(batch_start)
    def _():
        ldesc.append(pltpu.async_copy(learned_hbm.at[liota], learned_v, lsem))

    gds = [None] * _NSLOT
    wds = [None] * _NSLOT

    def fire(i):
        slot = i % _NSLOT
        if wds[slot] is not None:
            wds[slot].wait()
            wds[slot] = None
        vidx = tok_v[b, pl.ds(s0 + i * _L, _L)]
        gds[slot] = pltpu.async_copy(wte_hbm.at[vidx], rows_v.at[slot], gsem)

    def drain(i):
        slot = i % _NSLOT
        gds[slot].wait()
        wds[slot] = pltpu.async_copy(
            rows_v.at[slot], out_hbm.at[pl.ds(base + i * _L, _L)], osem)

    for i in range(_NBURST):
        fire(i)
        j = i - (_NSLOT - 1)
        if j >= 0:
            drain(j)
    for j in range(_NBURST - (_NSLOT - 1), _NBURST):
        drain(j)
    for w in wds:
        if w is not None:
            w.wait()

    @pl.when(batch_start)
    def _():
        ldesc[0].wait()
        svidx = liota + b * _S
        pltpu.async_copy(learned_v, out_hbm.at[svidx], lsem).wait()


def kernel(tokens, wte, learned_embedding):
    out = _soft_embed(tokens, wte, learned_embedding)
    return out.reshape(_B, _S, _D)


# speedup vs baseline: 1.0515x; 1.0180x over previous
import functools

import jax
import jax.numpy as jnp
from jax import lax
from jax.experimental import pallas as pl
from jax.experimental.pallas import tpu as pltpu
from jax.experimental.pallas import tpu_sc as plsc

_VOCAB = 100000
_D = 768
_N_TOK = 10
_B = 4
_S = 2048

_NC = 2
_NS = 16
_NW = _NC * _NS
_L = 16

_ROWS = _B * _S
_RPW = _ROWS // _NW
_NBURST = _RPW // _L
_NSLOT = 8
_WPB = _S // _RPW

_mesh = plsc.VectorSubcoreMesh(core_axis_name="c", subcore_axis_name="s")


@functools.partial(
    pl.kernel,
    mesh=_mesh,
    out_type=jax.ShapeDtypeStruct((_ROWS, _D), jnp.float32),
    scratch_types=[
        pltpu.VMEM((_B, _S), jnp.int32),
        pltpu.VMEM((_NSLOT, _L, _D), jnp.float32),
        pltpu.VMEM((_L, _D), jnp.float32),
        pltpu.SemaphoreType.DMA,
        pltpu.SemaphoreType.DMA,
        pltpu.SemaphoreType.DMA,
    ],
)
def _soft_embed(tokens_hbm, wte_hbm, learned_hbm, out_hbm,
                tok_v, rows_v, learned_v, gsem, osem, lsem):
    wid = lax.axis_index("c") * _NS + lax.axis_index("s")
    base = wid * _RPW
    b = wid // _WPB
    s0 = (wid % _WPB) * _RPW
    batch_start = base % _S == 0
    liota = jnp.minimum(lax.iota(jnp.int32, _L), _N_TOK - 1)

    pltpu.sync_copy(tokens_hbm, tok_v)

    ldesc = []

    @pl.when(batch_start)
    def _():
        ldesc.append(pltpu.async_copy(learned_hbm.at[liota], learned_v, lsem))

    gds = [None] * _NSLOT
    wds = [None] * _NSLOT

    def fire(i):
        slot = i % _NSLOT
        if wds[slot] is not None:
            wds[slot].wait()
            wds[slot] = None
        vidx = tok_v[b, pl.ds(s0 + i * _L, _L)]
        gds[slot] = pltpu.async_copy(wte_hbm.at[vidx], rows_v.at[slot], gsem)

    def drain(i):
        slot = i % _NSLOT
        gds[slot].wait()
        wds[slot] = pltpu.async_copy(
            rows_v.at[slot], out_hbm.at[pl.ds(base + i * _L, _L)], osem)

    for i in range(_NBURST):
        fire(i)
        j = i - (_NSLOT - 1)
        if j >= 0:
            drain(j)
    for j in range(_NBURST - (_NSLOT - 1), _NBURST):
        drain(j)
    for w in wds:
        if w is not None:
            w.wait()

    @pl.when(batch_start)
    def _():
        ldesc[0].wait()
        svidx = liota + b * _S
        pltpu.async_copy(learned_v, out_hbm.at[svidx], lsem).wait()


def kernel(tokens, wte, learned_embedding):
    out = _soft_embed(tokens, wte, learned_embedding)
    return out.reshape(_B, _S, _D)


# speedup vs baseline: 1.0574x; 1.0056x over previous
import functools

import jax
import jax.numpy as jnp
from jax import lax
from jax.experimental import pallas as pl
from jax.experimental.pallas import tpu as pltpu
from jax.experimental.pallas import tpu_sc as plsc

_VOCAB = 100000
_D = 768
_N_TOK = 10
_B = 4
_S = 2048

_NC = 2
_NS = 16
_NW = _NC * _NS
_L = 16

_ROWS = _B * _S
_RPW = _ROWS // _NW
_NBURST = _RPW // _L
_NSLOT = 8
_WPB = _S // _RPW

_mesh = plsc.VectorSubcoreMesh(core_axis_name="c", subcore_axis_name="s")


@functools.partial(
    pl.kernel,
    mesh=_mesh,
    out_type=jax.ShapeDtypeStruct((_ROWS, _D), jnp.float32),
    scratch_types=[
        pltpu.VMEM((_B, _S), jnp.int32),
        pltpu.VMEM((_NSLOT, _L, _D), jnp.float32),
        pltpu.VMEM((_L, _D), jnp.float32),
        pltpu.SemaphoreType.DMA,
        pltpu.SemaphoreType.DMA,
        pltpu.SemaphoreType.DMA,
    ],
)
def _soft_embed(tokens_hbm, wte_hbm, learned_hbm, out_hbm,
                tok_v, rows_v, learned_v, gsem, osem, lsem):
    wid = lax.axis_index("c") * _NS + lax.axis_index("s")
    base = wid * _RPW
    b = wid // _WPB
    s0 = (wid % _WPB) * _RPW
    batch_start = base % _S == 0
    liota = jnp.minimum(lax.iota(jnp.int32, _L), _N_TOK - 1)

    pltpu.sync_copy(tokens_hbm, tok_v)

    ldesc = []

    @pl.when(batch_start)
    def _():
        ldesc.append(pltpu.async_copy(learned_hbm.at[liota], learned_v, lsem))

    gds = [None] * _NSLOT
    wds = [None] * _NSLOT

    def fire(i):
        slot = i % _NSLOT
        if wds[slot] is not None:
            wds[slot].wait()
            wds[slot] = None
        vidx = tok_v[b, pl.ds(s0 + i * _L, _L)]
        gds[slot] = pltpu.async_copy(wte_hbm.at[vidx], rows_v.at[slot], gsem)

    def drain(i):
        slot = i % _NSLOT
        gds[slot].wait()
        wds[slot] = pltpu.async_copy(
            rows_v.at[slot], out_hbm.at[pl.ds(base + i * _L, _L)], osem)

    for i in range(_NBURST):
        fire(i)
        j = i - (_NSLOT - 1)
        if j >= 0:
            drain(j)
        if i == _NSLOT:
            @pl.when(batch_start)
            def _():
                ldesc[0].wait()
                svidx = liota + b * _S
                pltpu.async_copy(learned_v, out_hbm.at[svidx], lsem).wait()
    for j in range(_NBURST - (_NSLOT - 1), _NBURST):
        drain(j)
    for w in wds:
        if w is not None:
            w.wait()


def kernel(tokens, wte, learned_embedding):
    out = _soft_embed(tokens, wte, learned_embedding)
    return out.reshape(_B, _S, _D)


# speedup vs baseline: 1.0724x; 1.0142x over previous
import functools

import jax
import jax.numpy as jnp
from jax import lax
from jax.experimental import pallas as pl
from jax.experimental.pallas import tpu as pltpu
from jax.experimental.pallas import tpu_sc as plsc

_VOCAB = 100000
_D = 768
_N_TOK = 10
_B = 4
_S = 2048

_NC = 2
_NS = 16
_NW = _NC * _NS
_L = 16

_ROWS = _B * _S
_RPW = _ROWS // _NW
_NBURST = _RPW // _L
_NSLOT = 8
_WPB = _S // _RPW

_mesh = plsc.VectorSubcoreMesh(core_axis_name="c", subcore_axis_name="s")


@functools.partial(
    pl.kernel,
    mesh=_mesh,
    out_type=jax.ShapeDtypeStruct((_ROWS, _D), jnp.float32),
    scratch_types=[
        pltpu.VMEM((_B, _S), jnp.int32),
        pltpu.VMEM((_NSLOT, _L, _D), jnp.float32),
        pltpu.VMEM((_L, _D), jnp.float32),
        pltpu.SemaphoreType.DMA,
        pltpu.SemaphoreType.DMA,
        pltpu.SemaphoreType.DMA,
    ],
)
def _soft_embed(tokens_hbm, wte_hbm, learned_hbm, out_hbm,
                tok_v, rows_v, learned_v, gsem, osem, lsem):
    wid = lax.axis_index("c") * _NS + lax.axis_index("s")
    base = wid * _RPW
    b = wid // _WPB
    s0 = (wid % _WPB) * _RPW
    batch_start = base % _S == 0
    liota = jnp.minimum(lax.iota(jnp.int32, _L), _N_TOK - 1)

    pltpu.sync_copy(tokens_hbm, tok_v)

    ldesc = []

    @pl.when(batch_start)
    def _():
        ldesc.append(pltpu.async_copy(learned_hbm.at[liota], learned_v, lsem))

    gds = [None] * _NSLOT
    wds = [None] * _NSLOT

    def fire(i):
        slot = i % _NSLOT
        if wds[slot] is not None:
            wds[slot].wait()
            wds[slot] = None
        vidx = tok_v[b, pl.ds(s0 + i * _L, _L)]
        gds[slot] = pltpu.async_copy(wte_hbm.at[vidx], rows_v.at[slot], gsem)

    def drain(i):
        slot = i % _NSLOT
        gds[slot].wait()
        if i == 0:
            @pl.when(batch_start)
            def _():
                ldesc[0].wait()

                def body(c, _):
                    off = c * _L
                    for r in range(_N_TOK):
                        rows_v[slot, r, pl.ds(off, _L)] = (
                            learned_v[r, pl.ds(off, _L)])
                    return 0

                lax.fori_loop(0, _D // _L, body, 0)
        wds[slot] = pltpu.async_copy(
            rows_v.at[slot], out_hbm.at[pl.ds(base + i * _L, _L)], osem)

    for i in range(_NBURST):
        fire(i)
        j = i - (_NSLOT - 1)
        if j >= 0:
            drain(j)
    for j in range(_NBURST - (_NSLOT - 1), _NBURST):
        drain(j)
    for w in wds:
        if w is not None:
            w.wait()


def kernel(tokens, wte, learned_embedding):
    out = _soft_embed(tokens, wte, learned_embedding)
    return out.reshape(_B, _S, _D)


# speedup vs baseline: 1.1050x; 1.0304x over previous
import functools

import jax
import jax.numpy as jnp
from jax import lax
from jax.experimental import pallas as pl
from jax.experimental.pallas import tpu as pltpu
from jax.experimental.pallas import tpu_sc as plsc

_VOCAB = 100000
_D = 768
_N_TOK = 10
_B = 4
_S = 2048

_NC = 2
_NS = 16
_NW = _NC * _NS
_L = 16

_ROWS = _B * _S
_RPW = _ROWS // _NW
_NBURST = _RPW // _L
_NSLOT = 9
_WPB = _S // _RPW

_mesh = plsc.VectorSubcoreMesh(core_axis_name="c", subcore_axis_name="s")


@functools.partial(
    pl.kernel,
    mesh=_mesh,
    out_type=jax.ShapeDtypeStruct((_ROWS, _D), jnp.float32),
    scratch_types=[
        pltpu.VMEM((_RPW,), jnp.int32),
        pltpu.VMEM((_NSLOT, _L, _D), jnp.float32),
        pltpu.VMEM((_L, _D), jnp.float32),
        pltpu.SemaphoreType.DMA,
        pltpu.SemaphoreType.DMA,
        pltpu.SemaphoreType.DMA,
    ],
)
def _soft_embed(tokens_hbm, wte_hbm, learned_hbm, out_hbm,
                tok_v, rows_v, learned_v, gsem, osem, lsem):
    wid = lax.axis_index("c") * _NS + lax.axis_index("s")
    base = wid * _RPW
    b = wid // _WPB
    s0 = (wid % _WPB) * _RPW
    batch_start = base % _S == 0
    liota = jnp.minimum(lax.iota(jnp.int32, _L), _N_TOK - 1)

    pltpu.sync_copy(tokens_hbm.at[pl.ds(base, _RPW)], tok_v)

    ldesc = []

    @pl.when(batch_start)
    def _():
        ldesc.append(pltpu.async_copy(learned_hbm.at[liota], learned_v, lsem))

    gds = [None] * _NSLOT
    wds = [None] * _NSLOT

    def fire(i):
        slot = i % _NSLOT
        if wds[slot] is not None:
            wds[slot].wait()
            wds[slot] = None
        vidx = tok_v[pl.ds(i * _L, _L)]
        gds[slot] = pltpu.async_copy(wte_hbm.at[vidx], rows_v.at[slot], gsem)

    def drain(i):
        slot = i % _NSLOT
        gds[slot].wait()
        if i == 0:
            @pl.when(batch_start)
            def _():
                ldesc[0].wait()

                def body(c, _):
                    off = c * _L
                    for r in range(_N_TOK):
                        rows_v[slot, r, pl.ds(off, _L)] = (
                            learned_v[r, pl.ds(off, _L)])
                    return 0

                lax.fori_loop(0, _D // _L, body, 0)
        wds[slot] = pltpu.async_copy(
            rows_v.at[slot], out_hbm.at[pl.ds(base + i * _L, _L)], osem)

    for i in range(_NBURST):
        fire(i)
        j = i - (_NSLOT - 1)
        if j >= 0:
            drain(j)
    for j in range(_NBURST - (_NSLOT - 1), _NBURST):
        drain(j)
    for w in wds:
        if w is not None:
            w.wait()


def kernel(tokens, wte, learned_embedding):
    out = _soft_embed(tokens.reshape(_ROWS), wte, learned_embedding)
    return out.reshape(_B, _S, _D)


# speedup vs baseline: 1.1352x; 1.0274x over previous
import functools

import jax
import jax.numpy as jnp
from jax import lax
from jax.experimental import pallas as pl
from jax.experimental.pallas import tpu as pltpu
from jax.experimental.pallas import tpu_sc as plsc

_VOCAB = 100000
_D = 768
_N_TOK = 10
_B = 4
_S = 2048

_NC = 2
_NS = 16
_NW = _NC * _NS
_L = 16

_ROWS = _B * _S
_RPW = _ROWS // _NW
_NBURST = _RPW // _L
_NSLOT = 9
_WPB = _S // _RPW

_mesh = plsc.VectorSubcoreMesh(core_axis_name="c", subcore_axis_name="s")


@functools.partial(
    pl.kernel,
    mesh=_mesh,
    out_type=jax.ShapeDtypeStruct((_ROWS, _D), jnp.float32),
    scratch_types=[
        pltpu.VMEM((_RPW,), jnp.int32),
        pltpu.VMEM((_NSLOT, _L, _D), jnp.float32),
        pltpu.VMEM((_N_TOK, _D), jnp.float32),
        pltpu.SemaphoreType.DMA,
        pltpu.SemaphoreType.DMA,
        pltpu.SemaphoreType.DMA,
    ],
)
def _soft_embed(tokens_hbm, wte_hbm, learned_hbm, out_hbm,
                tok_v, rows_v, learned_v, gsem, osem, lsem):
    wid = lax.axis_index("c") * _NS + lax.axis_index("s")
    base = wid * _RPW
    batch_start = base % _S == 0

    pltpu.sync_copy(tokens_hbm.at[pl.ds(base, _RPW)], tok_v)

    ldesc = []

    @pl.when(batch_start)
    def _():
        ldesc.append(pltpu.async_copy(learned_hbm, learned_v, lsem))

    gds = [None] * _NSLOT
    wds = [None] * _NSLOT

    def fire(i):
        slot = i % _NSLOT
        if wds[slot] is not None:
            wds[slot].wait()
            wds[slot] = None
        vidx = tok_v[pl.ds(i * _L, _L)]
        gds[slot] = pltpu.async_copy(wte_hbm.at[vidx], rows_v.at[slot], gsem)

    def drain(i):
        slot = i % _NSLOT
        gds[slot].wait()
        if i == 0:
            @pl.when(batch_start)
            def _():
                ldesc[0].wait()

                def body(c, _):
                    off = c * _L
                    for r in range(_N_TOK):
                        rows_v[slot, r, pl.ds(off, _L)] = (
                            learned_v[r, pl.ds(off, _L)])
                    return 0

                lax.fori_loop(0, _D // _L, body, 0)
        wds[slot] = pltpu.async_copy(
            rows_v.at[slot], out_hbm.at[pl.ds(base + i * _L, _L)], osem)

    for i in range(_NBURST):
        fire(i)
        j = i - (_NSLOT - 1)
        if j >= 0:
            drain(j)
    for j in range(_NBURST - (_NSLOT - 1), _NBURST):
        drain(j)
    for w in wds:
        if w is not None:
            w.wait()


def kernel(tokens, wte, learned_embedding):
    out = _soft_embed(tokens.reshape(_ROWS), wte, learned_embedding)
    return out.reshape(_B, _S, _D)
